# Initial kernel scaffold; baseline (speedup 1.0000x reference)
import jax
import jax.numpy as jnp
from jax.experimental import pallas as pl


def kernel(Z, embeddings):
    raise NotImplementedError("write your pallas kernel here")


# speedup vs baseline: 1.6810x; 1.6810x over previous
import functools

import jax
import jax.numpy as jnp
from jax import lax
from jax.experimental import pallas as pl
from jax.experimental.pallas import tpu as pltpu
from jax.experimental.pallas import tpu_sc as plsc

N_ATOMS = 1_000_000
DIM = 128
CHUNK = 128
N_FULL = N_ATOMS // CHUNK
TAIL = N_ATOMS - N_FULL * CHUNK
NC = 2
NS = 16
NW = NC * NS
ITERS = -(-N_FULL // NW)

_mesh = plsc.VectorSubcoreMesh(core_axis_name="c", subcore_axis_name="s")


@functools.partial(
    pl.kernel,
    mesh=_mesh,
    out_type=jax.ShapeDtypeStruct((N_ATOMS, DIM), jnp.float32),
    scratch_types=[
        pltpu.VMEM((CHUNK,), jnp.int32),
        pltpu.VMEM((CHUNK, DIM), jnp.float32),
        pltpu.SemaphoreType.DMA,
    ],
)
def _embed(idx_hbm, table_hbm, out_hbm, idx_v, rows_v, sem):
    wid = lax.axis_index("s") * NC + lax.axis_index("c")

    def body(t, carry):
        chunk = wid + t * NW

        @pl.when(chunk < N_FULL)
        def _():
            base = chunk * CHUNK
            pltpu.sync_copy(idx_hbm.at[pl.ds(base, CHUNK)], idx_v)
            pltpu.async_copy(table_hbm.at[idx_v], rows_v, sem).wait()
            pltpu.sync_copy(rows_v, out_hbm.at[pl.ds(base, CHUNK)])

        return carry

    lax.fori_loop(0, ITERS, body, 0)

    @pl.when((wid == NW - 1) & (TAIL > 0))
    def _tail():
        base = N_FULL * CHUNK
        idx_t = idx_v.at[pl.ds(0, TAIL)]
        rows_t = rows_v.at[pl.ds(0, TAIL)]
        pltpu.sync_copy(idx_hbm.at[pl.ds(base, TAIL)], idx_t)
        pltpu.async_copy(table_hbm.at[idx_t], rows_t, sem).wait()
        pltpu.sync_copy(rows_t, out_hbm.at[pl.ds(base, TAIL)])


def kernel(Z, embeddings):
    return _embed(Z.astype(jnp.int32), embeddings)


# speedup vs baseline: 1.7174x; 1.0217x over previous
import functools

import jax
import jax.numpy as jnp
from jax import lax
from jax.experimental import pallas as pl
from jax.experimental.pallas import tpu as pltpu
from jax.experimental.pallas import tpu_sc as plsc

N_ATOMS = 1_000_000
DIM = 128
CHUNK = 128
N_FULL = N_ATOMS // CHUNK
TAIL = N_ATOMS - N_FULL * CHUNK
NC = 2
NS = 16
NW = NC * NS
BASE_CHUNKS = N_FULL // NW
EXTRA = N_FULL - BASE_CHUNKS * NW
NBUF = 5
ROUNDS = -(-(BASE_CHUNKS + 1) // NBUF)
SLAB = (BASE_CHUNKS + 1) * CHUNK

_mesh = plsc.VectorSubcoreMesh(core_axis_name="c", subcore_axis_name="s")


@functools.partial(
    pl.kernel,
    mesh=_mesh,
    out_type=jax.ShapeDtypeStruct((N_ATOMS, DIM), jnp.float32),
    scratch_types=[
        pltpu.VMEM((SLAB,), jnp.int32),
        pltpu.VMEM((NBUF, CHUNK, DIM), jnp.float32),
        pltpu.SemaphoreType.DMA,
    ]
    + [pltpu.SemaphoreType.DMA] * NBUF
    + [pltpu.SemaphoreType.DMA] * NBUF,
)
def _embed(idx_hbm, table_hbm, out_hbm, idx_v, rows_v, sem, *bsems):
    gsem = bsems[:NBUF]
    ssem = bsems[NBUF:]
    wid = lax.axis_index("s") * NC + lax.axis_index("c")
    n_my = jnp.where(wid < EXTRA, BASE_CHUNKS + 1, BASE_CHUNKS)
    start_chunk = wid * BASE_CHUNKS + jnp.minimum(wid, EXTRA)
    atom0 = start_chunk * CHUNK

    pltpu.sync_copy(
        idx_hbm.at[pl.ds(atom0, BASE_CHUNKS * CHUNK)],
        idx_v.at[pl.ds(0, BASE_CHUNKS * CHUNK)],
    )

    @pl.when(wid < EXTRA)
    def _extra_idx():
        pltpu.sync_copy(
            idx_hbm.at[pl.ds(atom0 + BASE_CHUNKS * CHUNK, CHUNK)],
            idx_v.at[pl.ds(BASE_CHUNKS * CHUNK, CHUNK)],
        )

    @pl.when(wid == NW - 1)
    def _tail_idx():
        pltpu.sync_copy(
            idx_hbm.at[pl.ds(N_FULL * CHUNK, TAIL)],
            idx_v.at[pl.ds(BASE_CHUNKS * CHUNK, TAIL)],
        )

    def _gather(v, b):
        pltpu.async_copy(
            table_hbm.at[idx_v.at[pl.ds(v * CHUNK, CHUNK)]], rows_v.at[b], gsem[b]
        )

    def _gather_wait(v, b):
        pltpu.make_async_copy(
            table_hbm.at[idx_v.at[pl.ds(v * CHUNK, CHUNK)]], rows_v.at[b], gsem[b]
        ).wait()

    def _store(v, b):
        pltpu.async_copy(
            rows_v.at[b], out_hbm.at[pl.ds((start_chunk + v) * CHUNK, CHUNK)], ssem[b]
        )

    def _store_wait(v, b):
        pltpu.make_async_copy(
            rows_v.at[b], out_hbm.at[pl.ds((start_chunk + v) * CHUNK, CHUNK)], ssem[b]
        ).wait()

    def round_body(r, carry):
        for b in range(NBUF):
            v = r * NBUF + b

            @pl.when(v < n_my)
            def _(b=b, v=v):
                @pl.when(r >= 1)
                def _wait_prev():
                    _store_wait(v - NBUF, b)

                _gather(v, b)

        for b in range(NBUF):
            v = r * NBUF + b

            @pl.when(v < n_my)
            def _(b=b, v=v):
                _gather_wait(v, b)
                _store(v, b)

        return carry

    lax.fori_loop(0, ROUNDS, round_body, 0)

    for b in range(NBUF):
        v = (ROUNDS - 1) * NBUF + b

        @pl.when(v < n_my)
        def _(b=b, v=v):
            _store_wait(v, b)

    @pl.when(wid == NW - 1)
    def _tail():
        base = N_FULL * CHUNK
        idx_t = idx_v.at[pl.ds(BASE_CHUNKS * CHUNK, TAIL)]
        rows_t = rows_v.at[0].at[pl.ds(0, TAIL)]
        pltpu.async_copy(table_hbm.at[idx_t], rows_t, sem).wait()
        pltpu.sync_copy(rows_t, out_hbm.at[pl.ds(base, TAIL)])


def kernel(Z, embeddings):
    return _embed(Z.astype(jnp.int32), embeddings)
